# baseline (speedup 1.0000x reference)
import functools

import jax
import jax.numpy as jnp
from jax import lax
from jax.experimental import pallas as pl
from jax.experimental.pallas import tpu as pltpu
from jax.experimental.pallas import tpu_sc as plsc

N = 10000
E = 160000
D = 256
H = D // 2
NC, NS, L = 2, 16, 16

CH = 64
CPT = 160
EPT = CH * CPT
E_PAD = EPT * NS
EPW = E_PAD // (NC * NS)
NP = 10240
NROW = N + 16
RPT = 624

_mesh = plsc.VectorSubcoreMesh(
    core_axis_name="c", subcore_axis_name="s", num_cores=NC, num_subcores=NS
)


_GDN = lax.GatherDimensionNumbers(
    offset_dims=(), collapsed_slice_dims=(0,), start_index_map=(0,)
)


def _lane_gather(v, idx):
    return lax.gather(
        v, idx[:, None], _GDN, slice_sizes=(1,),
        mode=lax.GatherScatterMode.PROMISE_IN_BOUNDS,
    )


@functools.partial(
    pl.kernel,
    out_type=(
        jax.ShapeDtypeStruct((NC * NP,), jnp.float32),
        jax.ShapeDtypeStruct((E_PAD,), jnp.int32),
        jax.ShapeDtypeStruct((E_PAD,), jnp.int32),
    ),
    mesh=_mesh,
    compiler_params=pltpu.CompilerParams(needs_layout_passes=False),
    scratch_types=[
        pltpu.VMEM((NS, NP // NS), jnp.float32),
        pltpu.VMEM_SHARED((NS, NP), jnp.float32),
    ],
)
def _deg_kernel(edges_hbm, dparts_hbm, ssrc_hbm, sdst_hbm, strip_v, board):
    c = lax.axis_index("c")
    s = lax.axis_index("s")
    w = c * NS + s

    def scoped(src_v, dst_v, cnt, ldeg, osrc, odst, posbuf):
        pltpu.sync_copy(edges_hbm.at[0, pl.ds(w * EPW, EPW)], src_v)
        pltpu.sync_copy(edges_hbm.at[1, pl.ds(w * EPW, EPW)], dst_v)

        zi = jnp.zeros((L,), jnp.int32)
        zf = jnp.zeros((L,), jnp.float32)

        def zero_body(k, _):
            cnt[pl.ds(k * L, L)] = zi
            ldeg[pl.ds(k * L, L)] = zf
            return 0

        lax.fori_loop(0, NP // L, zero_body, 0)

        onei = jnp.ones((L,), jnp.int32)
        onef = jnp.ones((L,), jnp.float32)

        def hist_body(k, _):
            plsc.addupdate_scatter(cnt, [src_v[pl.ds(k * L, L)]], onei)
            plsc.addupdate_scatter(ldeg, [dst_v[pl.ds(k * L, L)]], onef)
            return 0

        lax.fori_loop(0, EPW // L, hist_body, 0)

        def pfx_body(k, carry):
            v = cnt[pl.ds(k * L, L)]
            inc = plsc.cumsum(v)
            cnt[pl.ds(k * L, L)] = inc - v + carry
            return carry + jnp.sum(v)

        lax.fori_loop(0, NP // L, pfx_body, jnp.int32(0))

        lanes = lax.iota(jnp.int32, L)

        def place_body(k, _):
            sv = src_v[pl.ds(k * L, L)]
            dv = dst_v[pl.ds(k * L, L)]
            skey, slid = plsc.sort_key_val(sv, lanes)
            prev = _lane_gather(skey, jnp.maximum(lanes - 1, 0))
            nxt = _lane_gather(skey, jnp.minimum(lanes + 1, L - 1))
            is_start = (skey != prev) | (lanes == 0)
            is_last = (skey != nxt) | (lanes == L - 1)
            rank = lanes - plsc.cummax(jnp.where(is_start, lanes, 0))
            base = plsc.load_gather(cnt, [skey])
            pos_sorted = base + rank
            plsc.store_scatter(cnt, [skey], pos_sorted + 1, mask=is_last)
            plsc.store_scatter(posbuf, [slid], pos_sorted)
            pos = posbuf[...]
            plsc.store_scatter(osrc, [pos], sv)
            plsc.store_scatter(odst, [pos], dv)
            return 0

        lax.fori_loop(0, EPW // L, place_body, 0)

        pltpu.sync_copy(osrc, ssrc_hbm.at[pl.ds(w * EPW, EPW)])
        pltpu.sync_copy(odst, sdst_hbm.at[pl.ds(w * EPW, EPW)])

        pltpu.sync_copy(ldeg, board.at[s])
        plsc.subcore_barrier()

        w_cols = NP // NS
        pltpu.sync_copy(board.at[:, pl.ds(s * w_cols, w_cols)], strip_v)

        def red_body(k, _):
            v = strip_v[0, pl.ds(k * L, L)]
            for r in range(1, NS):
                v = v + strip_v[r, pl.ds(k * L, L)]
            strip_v[0, pl.ds(k * L, L)] = v
            return 0

        lax.fori_loop(0, w_cols // L, red_body, 0)
        pltpu.sync_copy(strip_v.at[0],
                        dparts_hbm.at[pl.ds(c * NP + s * w_cols, w_cols)])

    pl.run_scoped(
        scoped,
        pltpu.VMEM((EPW,), jnp.int32),
        pltpu.VMEM((EPW,), jnp.int32),
        pltpu.VMEM((NP,), jnp.int32),
        pltpu.VMEM((NP,), jnp.float32),
        pltpu.VMEM((EPW,), jnp.int32),
        pltpu.VMEM((EPW,), jnp.int32),
        pltpu.VMEM((L,), jnp.int32),
    )


@functools.partial(
    pl.kernel,
    out_type=jax.ShapeDtypeStruct((2, N, H), jnp.float32),
    mesh=_mesh,
    scratch_types=[
        pltpu.VMEM((EPT,), jnp.int32),
        pltpu.VMEM((CPT, CH), jnp.int32),
        pltpu.VMEM_SHARED((NROW, H), jnp.float32),
        pltpu.SemaphoreType.DMA,
        pltpu.SemaphoreType.DMA,
    ],
)
def _msg_kernel(p_hbm, src_hbm, dst_hbm, s_hbm, src_v, dst_v, acc,
                sem0, sem1):
    c = lax.axis_index("c")
    s = lax.axis_index("s")

    pltpu.sync_copy(src_hbm.at[pl.ds(s * EPT, EPT)], src_v)
    pltpu.sync_copy(dst_hbm.at[pl.ds(s * CPT, CPT), :], dst_v)

    def run(tab, osl, g0, g1):
        pltpu.sync_copy(tab.at[pl.ds(s * RPT, RPT)], acc.at[pl.ds(s * RPT, RPT)])

        @pl.when(s == NS - 1)
        def _():
            pltpu.sync_copy(tab.at[pl.ds(NS * RPT, N - NS * RPT)],
                            acc.at[pl.ds(NS * RPT, N - NS * RPT)])

        plsc.subcore_barrier()

        pltpu.async_copy(tab.at[src_v.at[pl.ds(0, CH)]], g0, sem0)

        def pair_body(i, _):
            j0 = 2 * i
            pltpu.async_copy(tab.at[src_v.at[pl.ds((j0 + 1) * CH, CH)]],
                             g1, sem1)
            pltpu.make_async_copy(tab.at[src_v.at[pl.ds(j0 * CH, CH)]],
                                  g0, sem0).wait()
            pltpu.sync_copy(g0, acc.at[dst_v.at[j0]], add=True)

            @pl.when(i < CPT // 2 - 1)
            def _():
                pltpu.async_copy(
                    tab.at[src_v.at[pl.ds((j0 + 2) * CH, CH)]], g0, sem0)

            pltpu.make_async_copy(tab.at[src_v.at[pl.ds((j0 + 1) * CH, CH)]],
                                  g1, sem1).wait()
            pltpu.sync_copy(g1, acc.at[dst_v.at[j0 + 1]], add=True)
            return 0

        lax.fori_loop(0, CPT // 2, pair_body, 0)

        plsc.subcore_barrier()
        pltpu.sync_copy(acc.at[pl.ds(s * RPT, RPT)], osl.at[pl.ds(s * RPT, RPT)])

        @pl.when(s == NS - 1)
        def _():
            pltpu.sync_copy(acc.at[pl.ds(NS * RPT, N - NS * RPT)],
                            osl.at[pl.ds(NS * RPT, N - NS * RPT)])

    def scoped(g0, g1):
        @pl.when(c == 0)
        def _():
            run(p_hbm.at[0], s_hbm.at[0], g0, g1)

        @pl.when(c == 1)
        def _():
            run(p_hbm.at[1], s_hbm.at[1], g0, g1)

    pl.run_scoped(scoped,
                  pltpu.VMEM((CH, H), jnp.float32),
                  pltpu.VMEM((CH, H), jnp.float32))


BR = 1000
NB = N // BR


def _dinv_of(deg_ref):
    dg = deg_ref[:, 0] + deg_ref[:, 1] + 1.0
    return lax.rsqrt(dg)


def _k2_body(x_ref, w_ref, deg_ref, o_ref):
    dinv = _dinv_of(deg_ref)
    h = jnp.dot(x_ref[...], w_ref[...], preferred_element_type=jnp.float32)
    o_ref[0] = h * dinv[:, None]


def _gelu_exact(t):
    return 0.5 * t * (1.0 + lax.erf(t * 0.7071067811865476))


def _k4_body(slo_ref, shi_ref, w_ref, b_ref, deg_ref, o_ref):
    dinv = _dinv_of(deg_ref)
    sblk = jnp.concatenate([slo_ref[0], shi_ref[0]], axis=1)
    t = sblk * dinv[:, None] + b_ref[...]
    g = _gelu_exact(t)
    h = jnp.dot(g, w_ref[...], preferred_element_type=jnp.float32)
    o_ref[0] = h * dinv[:, None]


def _k6_body(slo_ref, shi_ref, b_ref, deg_ref, o_ref):
    dinv = _dinv_of(deg_ref)
    o_ref[:, :H] = slo_ref[0] * dinv[:, None] + b_ref[:, :H]
    o_ref[:, H:] = shi_ref[0] * dinv[:, None] + b_ref[:, H:]


def _scale_matmul(x, w, deg2):
    return pl.pallas_call(
        _k2_body,
        grid=(2, NB),
        in_specs=[
            pl.BlockSpec((BR, D), lambda i, j: (j, 0)),
            pl.BlockSpec((D, H), lambda i, j: (0, i)),
            pl.BlockSpec((BR, 2), lambda i, j: (j, 0)),
        ],
        out_specs=pl.BlockSpec((1, BR, H), lambda i, j: (i, j, 0)),
        out_shape=jax.ShapeDtypeStruct((2, N, H), jnp.float32),
    )(x, w, deg2)


def _gelu_scale_matmul(s1, w, b, deg2):
    return pl.pallas_call(
        _k4_body,
        grid=(2, NB),
        in_specs=[
            pl.BlockSpec((1, BR, H), lambda i, j: (0, j, 0)),
            pl.BlockSpec((1, BR, H), lambda i, j: (1, j, 0)),
            pl.BlockSpec((D, H), lambda i, j: (0, i)),
            pl.BlockSpec((1, D), lambda i, j: (0, 0)),
            pl.BlockSpec((BR, 2), lambda i, j: (j, 0)),
        ],
        out_specs=pl.BlockSpec((1, BR, H), lambda i, j: (i, j, 0)),
        out_shape=jax.ShapeDtypeStruct((2, N, H), jnp.float32),
    )(s1, s1, w, b, deg2)


def _final_scale(s2, b, deg2):
    return pl.pallas_call(
        _k6_body,
        grid=(NB,),
        in_specs=[
            pl.BlockSpec((1, BR, H), lambda j: (0, j, 0)),
            pl.BlockSpec((1, BR, H), lambda j: (1, j, 0)),
            pl.BlockSpec((1, D), lambda j: (0, 0)),
            pl.BlockSpec((BR, 2), lambda j: (j, 0)),
        ],
        out_specs=pl.BlockSpec((BR, D), lambda j: (j, 0)),
        out_shape=jax.ShapeDtypeStruct((N, D), jnp.float32),
    )(s2, s2, b, deg2)


def kernel(x, edge_index, W1, b1, W2, b2):
    pad = jnp.broadcast_to(
        jnp.array([[0], [N]], dtype=jnp.int32), (2, E_PAD - E)
    )
    epad = jnp.concatenate([edge_index.astype(jnp.int32), pad], axis=1)
    b1r = b1.reshape(1, D)
    b2r = b2.reshape(1, D)

    dparts, ssrc, sdst = _deg_kernel(epad)
    src_flat = ssrc
    dst2d = sdst.reshape(E_PAD // CH, CH)
    deg2 = dparts.reshape(NC, NP)[:, :N].T

    p1 = _scale_matmul(x, W1, deg2)
    s1 = _msg_kernel(p1, src_flat, dst2d)
    p2 = _gelu_scale_matmul(s1, W2, b1r, deg2)
    s2 = _msg_kernel(p2, src_flat, dst2d)
    return _final_scale(s2, b2r, deg2)


# speedup vs baseline: 1.0531x; 1.0531x over previous
import functools

import jax
import jax.numpy as jnp
from jax import lax
from jax.experimental import pallas as pl
from jax.experimental.pallas import tpu as pltpu
from jax.experimental.pallas import tpu_sc as plsc

N = 10000
E = 160000
D = 256
H = D // 2
NC, NS, L = 2, 16, 16

CH = 64
CPT = 160
EPT = CH * CPT
E_PAD = EPT * NS
EPW = E_PAD // (NC * NS)
NP = 10240
NROW = N + 16
RPT = 624

_mesh = plsc.VectorSubcoreMesh(
    core_axis_name="c", subcore_axis_name="s", num_cores=NC, num_subcores=NS
)


@functools.partial(
    pl.kernel,
    out_type=jax.ShapeDtypeStruct((NC * NP,), jnp.float32),
    mesh=_mesh,
    compiler_params=pltpu.CompilerParams(needs_layout_passes=False),
    scratch_types=[
        pltpu.VMEM((EPW,), jnp.int32),
        pltpu.VMEM((NP,), jnp.float32),
        pltpu.VMEM((NS, NP // NS), jnp.float32),
        pltpu.VMEM_SHARED((NS, NP), jnp.float32),
    ],
)
def _deg_kernel(edges_hbm, dparts_hbm, dst_v, local_deg, strip_v, board):
    c = lax.axis_index("c")
    s = lax.axis_index("s")
    w = c * NS + s
    pltpu.sync_copy(edges_hbm.at[1, pl.ds(w * EPW, EPW)], dst_v)

    def zero_body(k, _):
        local_deg[pl.ds(k * L, L)] = jnp.zeros((L,), jnp.float32)
        return 0

    lax.fori_loop(0, NP // L, zero_body, 0)

    ones = jnp.ones((L,), jnp.float32)

    def acc_body(k, _):
        idx = dst_v[pl.ds(k * L, L)]
        plsc.addupdate_scatter(local_deg, [idx], ones)
        return 0

    lax.fori_loop(0, EPW // L, acc_body, 0)

    pltpu.sync_copy(local_deg, board.at[s])
    plsc.subcore_barrier()

    w_cols = NP // NS
    pltpu.sync_copy(board.at[:, pl.ds(s * w_cols, w_cols)], strip_v)

    def red_body(k, _):
        v = strip_v[0, pl.ds(k * L, L)]
        for r in range(1, NS):
            v = v + strip_v[r, pl.ds(k * L, L)]
        strip_v[0, pl.ds(k * L, L)] = v
        return 0

    lax.fori_loop(0, w_cols // L, red_body, 0)
    pltpu.sync_copy(strip_v.at[0], dparts_hbm.at[pl.ds(c * NP + s * w_cols, w_cols)])


@functools.partial(
    pl.kernel,
    out_type=jax.ShapeDtypeStruct((2, N, H), jnp.float32),
    mesh=_mesh,
    scratch_types=[
        pltpu.VMEM((EPT,), jnp.int32),
        pltpu.VMEM((CPT, CH), jnp.int32),
        pltpu.VMEM_SHARED((NROW, H), jnp.float32),
        pltpu.SemaphoreType.DMA,
        pltpu.SemaphoreType.DMA,
    ],
)
def _msg_kernel(p_hbm, src_hbm, dst_hbm, s_hbm, src_v, dst_v, acc,
                sem0, sem1):
    c = lax.axis_index("c")
    s = lax.axis_index("s")

    pltpu.sync_copy(src_hbm.at[pl.ds(s * EPT, EPT)], src_v)
    pltpu.sync_copy(dst_hbm.at[pl.ds(s * CPT, CPT), :], dst_v)

    def run(tab, osl, g0, g1):
        pltpu.sync_copy(tab.at[pl.ds(s * RPT, RPT)], acc.at[pl.ds(s * RPT, RPT)])

        @pl.when(s == NS - 1)
        def _():
            pltpu.sync_copy(tab.at[pl.ds(NS * RPT, N - NS * RPT)],
                            acc.at[pl.ds(NS * RPT, N - NS * RPT)])

        plsc.subcore_barrier()

        pltpu.async_copy(tab.at[src_v.at[pl.ds(0, CH)]], g0, sem0)

        def pair_body(i, _):
            j0 = 2 * i
            pltpu.async_copy(tab.at[src_v.at[pl.ds((j0 + 1) * CH, CH)]],
                             g1, sem1)
            pltpu.make_async_copy(tab.at[src_v.at[pl.ds(j0 * CH, CH)]],
                                  g0, sem0).wait()
            pltpu.sync_copy(g0, acc.at[dst_v.at[j0]], add=True)

            @pl.when(i < CPT // 2 - 1)
            def _():
                pltpu.async_copy(
                    tab.at[src_v.at[pl.ds((j0 + 2) * CH, CH)]], g0, sem0)

            pltpu.make_async_copy(tab.at[src_v.at[pl.ds((j0 + 1) * CH, CH)]],
                                  g1, sem1).wait()
            pltpu.sync_copy(g1, acc.at[dst_v.at[j0 + 1]], add=True)
            return 0

        lax.fori_loop(0, CPT // 2, pair_body, 0)

        plsc.subcore_barrier()
        pltpu.sync_copy(acc.at[pl.ds(s * RPT, RPT)], osl.at[pl.ds(s * RPT, RPT)])

        @pl.when(s == NS - 1)
        def _():
            pltpu.sync_copy(acc.at[pl.ds(NS * RPT, N - NS * RPT)],
                            osl.at[pl.ds(NS * RPT, N - NS * RPT)])

    def scoped(g0, g1):
        @pl.when(c == 0)
        def _():
            run(p_hbm.at[0], s_hbm.at[0], g0, g1)

        @pl.when(c == 1)
        def _():
            run(p_hbm.at[1], s_hbm.at[1], g0, g1)

    pl.run_scoped(scoped,
                  pltpu.VMEM((CH, H), jnp.float32),
                  pltpu.VMEM((CH, H), jnp.float32))


BR = 1000
NB = N // BR


def _dinv_of(deg_ref):
    dg = deg_ref[:, 0] + deg_ref[:, 1] + 1.0
    return lax.rsqrt(dg)


def _k2_body(x_ref, w_ref, deg_ref, o_ref):
    dinv = _dinv_of(deg_ref)
    h = jnp.dot(x_ref[...], w_ref[...], preferred_element_type=jnp.float32)
    o_ref[0] = h * dinv[:, None]


def _gelu_exact(t):
    return 0.5 * t * (1.0 + lax.erf(t * 0.7071067811865476))


def _k4_body(slo_ref, shi_ref, w_ref, b_ref, deg_ref, o_ref):
    dinv = _dinv_of(deg_ref)
    sblk = jnp.concatenate([slo_ref[0], shi_ref[0]], axis=1)
    t = sblk * dinv[:, None] + b_ref[...]
    g = _gelu_exact(t)
    h = jnp.dot(g, w_ref[...], preferred_element_type=jnp.float32)
    o_ref[0] = h * dinv[:, None]


def _k6_body(slo_ref, shi_ref, b_ref, deg_ref, o_ref):
    dinv = _dinv_of(deg_ref)
    o_ref[:, :H] = slo_ref[0] * dinv[:, None] + b_ref[:, :H]
    o_ref[:, H:] = shi_ref[0] * dinv[:, None] + b_ref[:, H:]


def _scale_matmul(x, w, deg2):
    return pl.pallas_call(
        _k2_body,
        grid=(2, NB),
        in_specs=[
            pl.BlockSpec((BR, D), lambda i, j: (j, 0)),
            pl.BlockSpec((D, H), lambda i, j: (0, i)),
            pl.BlockSpec((BR, 2), lambda i, j: (j, 0)),
        ],
        out_specs=pl.BlockSpec((1, BR, H), lambda i, j: (i, j, 0)),
        out_shape=jax.ShapeDtypeStruct((2, N, H), jnp.float32),
    )(x, w, deg2)


def _gelu_scale_matmul(s1, w, b, deg2):
    return pl.pallas_call(
        _k4_body,
        grid=(2, NB),
        in_specs=[
            pl.BlockSpec((1, BR, H), lambda i, j: (0, j, 0)),
            pl.BlockSpec((1, BR, H), lambda i, j: (1, j, 0)),
            pl.BlockSpec((D, H), lambda i, j: (0, i)),
            pl.BlockSpec((1, D), lambda i, j: (0, 0)),
            pl.BlockSpec((BR, 2), lambda i, j: (j, 0)),
        ],
        out_specs=pl.BlockSpec((1, BR, H), lambda i, j: (i, j, 0)),
        out_shape=jax.ShapeDtypeStruct((2, N, H), jnp.float32),
    )(s1, s1, w, b, deg2)


def _final_scale(s2, b, deg2):
    return pl.pallas_call(
        _k6_body,
        grid=(NB,),
        in_specs=[
            pl.BlockSpec((1, BR, H), lambda j: (0, j, 0)),
            pl.BlockSpec((1, BR, H), lambda j: (1, j, 0)),
            pl.BlockSpec((1, D), lambda j: (0, 0)),
            pl.BlockSpec((BR, 2), lambda j: (j, 0)),
        ],
        out_specs=pl.BlockSpec((BR, D), lambda j: (j, 0)),
        out_shape=jax.ShapeDtypeStruct((N, D), jnp.float32),
    )(s2, s2, b, deg2)


def kernel(x, edge_index, W1, b1, W2, b2):
    pad = jnp.broadcast_to(
        jnp.array([[0], [N]], dtype=jnp.int32), (2, E_PAD - E)
    )
    epad = jnp.concatenate([edge_index.astype(jnp.int32), pad], axis=1)
    src_flat = epad[0]
    dst2d = epad[1].reshape(E_PAD // CH, CH)
    b1r = b1.reshape(1, D)
    b2r = b2.reshape(1, D)

    dparts = _deg_kernel(epad)
    deg2 = dparts.reshape(NC, NP)[:, :N].T

    p1 = _scale_matmul(x, W1, deg2)
    s1 = _msg_kernel(p1, src_flat, dst2d)
    p2 = _gelu_scale_matmul(s1, W2, b1r, deg2)
    s2 = _msg_kernel(p2, src_flat, dst2d)
    return _final_scale(s2, b2r, deg2)


# speedup vs baseline: 1.1066x; 1.0508x over previous
import functools

import jax
import jax.numpy as jnp
from jax import lax
from jax.experimental import pallas as pl
from jax.experimental.pallas import tpu as pltpu
from jax.experimental.pallas import tpu_sc as plsc

N = 10000
E = 160000
D = 256
H = D // 2
NC, NS, L = 2, 16, 16

CH = 80
CPT = 128
EPT = CH * CPT
E_PAD = EPT * NS
EPW = E_PAD // (NC * NS)
NP = 10240
NROW = N + 16
RPT = 624

_mesh = plsc.VectorSubcoreMesh(
    core_axis_name="c", subcore_axis_name="s", num_cores=NC, num_subcores=NS
)


@functools.partial(
    pl.kernel,
    out_type=jax.ShapeDtypeStruct((NC * NP,), jnp.float32),
    mesh=_mesh,
    compiler_params=pltpu.CompilerParams(needs_layout_passes=False),
    scratch_types=[
        pltpu.VMEM((EPW,), jnp.int32),
        pltpu.VMEM((NP,), jnp.float32),
        pltpu.VMEM((NS, NP // NS), jnp.float32),
        pltpu.VMEM_SHARED((NS, NP), jnp.float32),
    ],
)
def _deg_kernel(edges_hbm, dparts_hbm, dst_v, local_deg, strip_v, board):
    c = lax.axis_index("c")
    s = lax.axis_index("s")
    w = c * NS + s
    pltpu.sync_copy(edges_hbm.at[1, pl.ds(w * EPW, EPW)], dst_v)

    def zero_body(k, _):
        local_deg[pl.ds(k * L, L)] = jnp.zeros((L,), jnp.float32)
        return 0

    lax.fori_loop(0, NP // L, zero_body, 0)

    ones = jnp.ones((L,), jnp.float32)

    def acc_body(k, _):
        idx = dst_v[pl.ds(k * L, L)]
        plsc.addupdate_scatter(local_deg, [idx], ones)
        return 0

    lax.fori_loop(0, EPW // L, acc_body, 0)

    pltpu.sync_copy(local_deg, board.at[s])
    plsc.subcore_barrier()

    w_cols = NP // NS
    pltpu.sync_copy(board.at[:, pl.ds(s * w_cols, w_cols)], strip_v)

    def red_body(k, _):
        v = strip_v[0, pl.ds(k * L, L)]
        for r in range(1, NS):
            v = v + strip_v[r, pl.ds(k * L, L)]
        strip_v[0, pl.ds(k * L, L)] = v
        return 0

    lax.fori_loop(0, w_cols // L, red_body, 0)
    pltpu.sync_copy(strip_v.at[0], dparts_hbm.at[pl.ds(c * NP + s * w_cols, w_cols)])


@functools.partial(
    pl.kernel,
    out_type=jax.ShapeDtypeStruct((2, N, H), jnp.float32),
    mesh=_mesh,
    scratch_types=[
        pltpu.VMEM((EPT,), jnp.int32),
        pltpu.VMEM((CPT, CH), jnp.int32),
        pltpu.VMEM_SHARED((NROW, H), jnp.float32),
        pltpu.SemaphoreType.DMA,
        pltpu.SemaphoreType.DMA,
    ],
)
def _msg_kernel(p_hbm, src_hbm, dst_hbm, s_hbm, src_v, dst_v, acc,
                sem0, sem1):
    c = lax.axis_index("c")
    s = lax.axis_index("s")

    pltpu.sync_copy(src_hbm.at[pl.ds(s * EPT, EPT)], src_v)
    pltpu.sync_copy(dst_hbm.at[pl.ds(s * CPT, CPT), :], dst_v)

    def run(tab, osl, g0, g1):
        pltpu.sync_copy(tab.at[pl.ds(s * RPT, RPT)], acc.at[pl.ds(s * RPT, RPT)])

        @pl.when(s == NS - 1)
        def _():
            pltpu.sync_copy(tab.at[pl.ds(NS * RPT, N - NS * RPT)],
                            acc.at[pl.ds(NS * RPT, N - NS * RPT)])

        plsc.subcore_barrier()

        pltpu.async_copy(tab.at[src_v.at[pl.ds(0, CH)]], g0, sem0)

        def pair_body(i, _):
            j0 = 2 * i
            pltpu.async_copy(tab.at[src_v.at[pl.ds((j0 + 1) * CH, CH)]],
                             g1, sem1)
            pltpu.make_async_copy(tab.at[src_v.at[pl.ds(j0 * CH, CH)]],
                                  g0, sem0).wait()
            pltpu.sync_copy(g0, acc.at[dst_v.at[j0]], add=True)

            @pl.when(i < CPT // 2 - 1)
            def _():
                pltpu.async_copy(
                    tab.at[src_v.at[pl.ds((j0 + 2) * CH, CH)]], g0, sem0)

            pltpu.make_async_copy(tab.at[src_v.at[pl.ds((j0 + 1) * CH, CH)]],
                                  g1, sem1).wait()
            pltpu.sync_copy(g1, acc.at[dst_v.at[j0 + 1]], add=True)
            return 0

        lax.fori_loop(0, CPT // 2, pair_body, 0)

        plsc.subcore_barrier()
        pltpu.sync_copy(acc.at[pl.ds(s * RPT, RPT)], osl.at[pl.ds(s * RPT, RPT)])

        @pl.when(s == NS - 1)
        def _():
            pltpu.sync_copy(acc.at[pl.ds(NS * RPT, N - NS * RPT)],
                            osl.at[pl.ds(NS * RPT, N - NS * RPT)])

    def scoped(g0, g1):
        @pl.when(c == 0)
        def _():
            run(p_hbm.at[0], s_hbm.at[0], g0, g1)

        @pl.when(c == 1)
        def _():
            run(p_hbm.at[1], s_hbm.at[1], g0, g1)

    pl.run_scoped(scoped,
                  pltpu.VMEM((CH, H), jnp.float32),
                  pltpu.VMEM((CH, H), jnp.float32))


BR = 1000
NB = N // BR


def _dinv_of(deg_ref):
    dg = deg_ref[:, 0] + deg_ref[:, 1] + 1.0
    return lax.rsqrt(dg)


def _k2_body(x_ref, w_ref, deg_ref, o_ref):
    dinv = _dinv_of(deg_ref)
    h = jnp.dot(x_ref[...], w_ref[...], preferred_element_type=jnp.float32)
    o_ref[0] = h * dinv[:, None]


def _gelu_exact(t):
    return 0.5 * t * (1.0 + lax.erf(t * 0.7071067811865476))


def _k4_body(slo_ref, shi_ref, w_ref, b_ref, deg_ref, o_ref):
    dinv = _dinv_of(deg_ref)
    sblk = jnp.concatenate([slo_ref[0], shi_ref[0]], axis=1)
    t = sblk * dinv[:, None] + b_ref[...]
    g = _gelu_exact(t)
    h = jnp.dot(g, w_ref[...], preferred_element_type=jnp.float32)
    o_ref[0] = h * dinv[:, None]


def _k6_body(slo_ref, shi_ref, b_ref, deg_ref, o_ref):
    dinv = _dinv_of(deg_ref)
    o_ref[:, :H] = slo_ref[0] * dinv[:, None] + b_ref[:, :H]
    o_ref[:, H:] = shi_ref[0] * dinv[:, None] + b_ref[:, H:]


def _scale_matmul(x, w, deg2):
    return pl.pallas_call(
        _k2_body,
        grid=(2, NB),
        in_specs=[
            pl.BlockSpec((BR, D), lambda i, j: (j, 0)),
            pl.BlockSpec((D, H), lambda i, j: (0, i)),
            pl.BlockSpec((BR, 2), lambda i, j: (j, 0)),
        ],
        out_specs=pl.BlockSpec((1, BR, H), lambda i, j: (i, j, 0)),
        out_shape=jax.ShapeDtypeStruct((2, N, H), jnp.float32),
    )(x, w, deg2)


def _gelu_scale_matmul(s1, w, b, deg2):
    return pl.pallas_call(
        _k4_body,
        grid=(2, NB),
        in_specs=[
            pl.BlockSpec((1, BR, H), lambda i, j: (0, j, 0)),
            pl.BlockSpec((1, BR, H), lambda i, j: (1, j, 0)),
            pl.BlockSpec((D, H), lambda i, j: (0, i)),
            pl.BlockSpec((1, D), lambda i, j: (0, 0)),
            pl.BlockSpec((BR, 2), lambda i, j: (j, 0)),
        ],
        out_specs=pl.BlockSpec((1, BR, H), lambda i, j: (i, j, 0)),
        out_shape=jax.ShapeDtypeStruct((2, N, H), jnp.float32),
    )(s1, s1, w, b, deg2)


def _final_scale(s2, b, deg2):
    return pl.pallas_call(
        _k6_body,
        grid=(NB,),
        in_specs=[
            pl.BlockSpec((1, BR, H), lambda j: (0, j, 0)),
            pl.BlockSpec((1, BR, H), lambda j: (1, j, 0)),
            pl.BlockSpec((1, D), lambda j: (0, 0)),
            pl.BlockSpec((BR, 2), lambda j: (j, 0)),
        ],
        out_specs=pl.BlockSpec((BR, D), lambda j: (j, 0)),
        out_shape=jax.ShapeDtypeStruct((N, D), jnp.float32),
    )(s2, s2, b, deg2)


def kernel(x, edge_index, W1, b1, W2, b2):
    pad = jnp.broadcast_to(
        jnp.array([[0], [N]], dtype=jnp.int32), (2, E_PAD - E)
    )
    epad = jnp.concatenate([edge_index.astype(jnp.int32), pad], axis=1)
    src_flat = epad[0]
    dst2d = epad[1].reshape(E_PAD // CH, CH)
    b1r = b1.reshape(1, D)
    b2r = b2.reshape(1, D)

    dparts = _deg_kernel(epad)
    deg2 = dparts.reshape(NC, NP)[:, :N].T

    p1 = _scale_matmul(x, W1, deg2)
    s1 = _msg_kernel(p1, src_flat, dst2d)
    p2 = _gelu_scale_matmul(s1, W2, b1r, deg2)
    s2 = _msg_kernel(p2, src_flat, dst2d)
    return _final_scale(s2, b2r, deg2)
